# baseline (speedup 1.0000x reference)
import functools

import jax
import jax.numpy as jnp
from jax import lax
from jax.experimental import pallas as pl
from jax.experimental.pallas import tpu as pltpu
from jax.experimental.pallas import tpu_sc as plsc

B = 4096
CTX = 20
D = 128
V = 100000

NC = 2
NS = 16
NW = NC * NS
BPW = B // NW
CHUNK = 16
NCH = BPW // CHUNK


def _mean_emb_sc(idx_flat, table):
    mesh = plsc.VectorSubcoreMesh(core_axis_name="c", subcore_axis_name="s")

    @functools.partial(
        pl.kernel,
        mesh=mesh,
        out_type=jax.ShapeDtypeStruct((B, D), jnp.float32),
        scratch_types=[
            pltpu.VMEM((CHUNK * CTX,), jnp.int32),
            pltpu.VMEM((CHUNK * CTX, D), jnp.float32),
            pltpu.VMEM((CHUNK, D), jnp.float32),
            pltpu.SemaphoreType.DMA,
        ],
    )
    def sc_kernel(idx_hbm, table_hbm, out_hbm, idx_v, rows_v, acc_v, sem):
        wid = lax.axis_index("s") * NC + lax.axis_index("c")
        base = wid * BPW

        def chunk_body(ch, carry):
            row0 = base + ch * CHUNK
            pltpu.sync_copy(idx_hbm.at[pl.ds(row0 * CTX, CHUNK * CTX)], idx_v)
            pltpu.async_copy(table_hbm.at[idx_v], rows_v, sem).wait()

            def b_body(b, carry2):
                for j in range(D // 16):
                    acc = jnp.zeros((16,), jnp.float32)
                    for c in range(CTX):
                        acc = acc + rows_v[b * CTX + c, pl.ds(j * 16, 16)]
                    acc_v[b, pl.ds(j * 16, 16)] = acc * (1.0 / CTX)
                return carry2

            lax.fori_loop(0, CHUNK, b_body, 0)
            pltpu.sync_copy(acc_v, out_hbm.at[pl.ds(row0, CHUNK)])
            return carry

        lax.fori_loop(0, NCH, chunk_body, 0)

    return sc_kernel(idx_flat, table)


BN = 1024


def _project_tc(mean_emb, lin_w, lin_b2d):
    def mm_kernel(m_ref, w_ref, b_ref, o_ref):
        o_ref[...] = lax.dot_general(
            m_ref[...], w_ref[...],
            (((1,), (1,)), ((), ())),
            preferred_element_type=jnp.float32,
        ) + b_ref[...]

    return pl.pallas_call(
        mm_kernel,
        grid=(pl.cdiv(V, BN),),
        in_specs=[
            pl.BlockSpec((B, D), lambda j: (0, 0)),
            pl.BlockSpec((BN, D), lambda j: (j, 0)),
            pl.BlockSpec((1, BN), lambda j: (0, j)),
        ],
        out_specs=pl.BlockSpec((B, BN), lambda j: (0, j)),
        out_shape=jax.ShapeDtypeStruct((B, V), jnp.float32),
    )(mean_emb, lin_w, lin_b2d)


def kernel(context_words, emb_table, lin_w, lin_b):
    idx_flat = context_words.reshape(-1).astype(jnp.int32)
    mean_emb = _mean_emb_sc(idx_flat, emb_table)
    return _project_tc(mean_emb, lin_w, lin_b.reshape(1, V))
